# baseline (speedup 1.0000x reference)
import functools

import jax
import jax.numpy as jnp
from jax import lax
from jax.experimental import pallas as pl
from jax.experimental.pallas import tpu as pltpu
from jax.experimental.pallas import tpu_sc as plsc

_CCH = 256
_L = 16
_TS = 128

_PARAMS = dict(use_tc_tiling_on_sc=True, needs_layout_passes=False)


def _sc_transpose(V, D, NC, NS):
    NW = NC * NS
    V0 = (V // _TS) * _TS
    TAIL = V - V0
    n_strips = V0 // _TS
    per = n_strips // NW
    extra = n_strips - per * NW
    assert extra % 2 == 0 and per % 2 == 0 and D % _L == 0
    assert 0 < TAIL and TAIL % 8 == 0
    n_pairs = (per + 2) // 2
    mesh = plsc.VectorSubcoreMesh(core_axis_name="c", subcore_axis_name="s")

    @functools.partial(
        pl.kernel,
        out_type=jax.ShapeDtypeStruct((V, D), jnp.float32),
        mesh=mesh,
        compiler_params=pltpu.CompilerParams(**_PARAMS),
        scratch_types=[
            pltpu.VMEM((2, D, _TS), jnp.float32),
            pltpu.VMEM((2, _TS, D), jnp.float32),
            pltpu.VMEM((TAIL, D), jnp.float32),
            pltpu.SemaphoreType.DMA,
            pltpu.SemaphoreType.DMA,
            pltpu.SemaphoreType.DMA,
            pltpu.SemaphoreType.DMA,
        ],
    )
    def body(tabT, tail_hbm, tab_out, src_v, dst_v, tail_v, si0, si1, so0, so1):
        wid = lax.axis_index("s") * NC + lax.axis_index("c")
        nx = extra // 2
        cnt = jnp.where(wid < nx, per + 2, per)
        base = wid * per + 2 * jnp.minimum(wid, nx)
        sin = (si0, si1)
        sout = (so0, so1)

        def off(s):
            return pl.multiple_of((base + s) * _TS, _TS)

        def fetch(s, buf):
            pltpu.async_copy(tabT.at[:, pl.ds(off(s), _TS)], src_v.at[buf], sin[buf])

        def wait_fetch(s, buf):
            pltpu.make_async_copy(
                tabT.at[:, pl.ds(off(s), _TS)], src_v.at[buf], sin[buf]
            ).wait()

        def put(s, buf):
            pltpu.async_copy(dst_v.at[buf], tab_out.at[pl.ds(off(s), _TS)], sout[buf])

        def wait_put(s, buf):
            pltpu.make_async_copy(
                dst_v.at[buf], tab_out.at[pl.ds(off(s), _TS)], sout[buf]
            ).wait()

        def transpose(buf):
            for l in range(_TS):
                lane = jnp.full((_L,), l, jnp.int32)
                for h in range(D // _L):
                    cvec = lax.iota(jnp.int32, _L) + h * _L
                    val = plsc.load_gather(src_v.at[buf], [cvec, lane])
                    dst_v[buf, l, pl.ds(h * _L, _L)] = val

        fetch(0, 0)
        fetch(1, 1)

        def halfstep(s, buf):
            pl.when(s < cnt)(lambda: wait_fetch(s, buf))
            pl.when((s >= 2) & (s - 2 < cnt))(lambda: wait_put(s - 2, buf))
            transpose(buf)
            pl.when(s < cnt)(lambda: put(s, buf))
            pl.when(s + 2 < cnt)(lambda: fetch(s + 2, buf))

        def step(p, carry):
            halfstep(2 * p, 0)
            halfstep(2 * p + 1, 1)
            return carry

        lax.fori_loop(0, n_pairs, step, 0)
        pl.when(wid < nx)(lambda: wait_put(per, 0))
        pl.when(wid < nx)(lambda: wait_put(per + 1, 1))

        @pl.when(wid == NW - 1)
        def _():
            pltpu.sync_copy(tail_hbm, tail_v)
            pltpu.sync_copy(tail_v, tab_out.at[pl.ds(V0, TAIL)])

    return body


def _sc_gather_c(N, D, NC, NS):
    NW = NC * NS
    bc = N // NW
    n_ch = bc // _CCH
    assert bc % _CCH == 0 and n_ch % 2 == 0 and n_ch >= 4
    mesh = plsc.VectorSubcoreMesh(core_axis_name="c", subcore_axis_name="s")

    @functools.partial(
        pl.kernel,
        out_type=jax.ShapeDtypeStruct((N, D), jnp.float32),
        mesh=mesh,
        compiler_params=pltpu.CompilerParams(**_PARAMS),
        scratch_types=[
            pltpu.VMEM((bc,), jnp.int32),
            pltpu.VMEM((2, _CCH, D), jnp.float32),
            pltpu.SemaphoreType.DMA,
            pltpu.SemaphoreType.DMA,
        ],
    )
    def body(cidx_hbm, ctab, c_out, cidx_v, rows_v, sem0, sem1):
        wid = lax.axis_index("s") * NC + lax.axis_index("c")
        pltpu.sync_copy(cidx_hbm.at[wid], cidx_v)
        c_base = wid * bc
        sems = (sem0, sem1)

        def issue(chunk, buf):
            def blk(j, carry):
                v16 = cidx_v[pl.ds(chunk * _CCH + j * _L, _L)]
                for k in range(_L):
                    pltpu.async_copy(
                        ctab.at[pl.ds(v16[k], 1)],
                        rows_v.at[buf, pl.ds(j * _L + k, 1)],
                        sems[buf],
                    )
                return carry

            lax.fori_loop(0, _CCH // _L, blk, 0)

        def drain_write(chunk, buf):
            pltpu.make_async_copy(
                c_out.at[pl.ds(0, _CCH)], rows_v.at[buf], sems[buf]
            ).wait()
            pltpu.sync_copy(
                rows_v.at[buf], c_out.at[pl.ds(c_base + chunk * _CCH, _CCH)]
            )

        issue(0, 0)
        issue(1, 1)

        def step(p, carry):
            drain_write(2 * p, 0)
            issue(2 * p + 2, 0)
            drain_write(2 * p + 1, 1)
            issue(2 * p + 3, 1)
            return carry

        lax.fori_loop(0, n_ch // 2 - 1, step, 0)
        drain_write(n_ch - 2, 0)
        drain_write(n_ch - 1, 1)

    return body


def _sc_gather_w(B, D, V, NC, NS):
    NW = NC * NS
    bw = B // NW
    assert bw % _L == 0
    mesh = plsc.VectorSubcoreMesh(core_axis_name="c", subcore_axis_name="s")

    @functools.partial(
        pl.kernel,
        out_type=jax.ShapeDtypeStruct((B, D), jnp.float32),
        mesh=mesh,
        compiler_params=pltpu.CompilerParams(**_PARAMS),
        scratch_types=[
            pltpu.VMEM((bw,), jnp.int32),
            pltpu.VMEM((bw, D), jnp.float32),
            pltpu.SemaphoreType.DMA,
        ],
    )
    def body(widx_hbm, wtab, w_out, widx_v, rows_v, sem):
        wid = lax.axis_index("s") * NC + lax.axis_index("c")
        pltpu.sync_copy(widx_hbm.at[wid], widx_v)

        def issue(j, carry):
            v16 = widx_v[pl.ds(j * _L, _L)]
            for k in range(_L):
                pltpu.async_copy(
                    wtab.at[pl.ds(v16[k], 1)],
                    rows_v.at[pl.ds(j * _L + k, 1)],
                    sem,
                )
            return carry

        lax.fori_loop(0, bw // _L, issue, 0)
        pltpu.make_async_copy(w_out.at[pl.ds(0, bw)], rows_v, sem).wait()
        pltpu.sync_copy(rows_v, w_out.at[pl.ds(wid * bw, bw)])

    return body


def kernel(words, contexts, w_embedding, c_embedding):
    (B,) = words.shape
    _, CTX = contexts.shape
    V, D = w_embedding.shape
    N = B * CTX
    info = plsc.get_sparse_core_info()
    NC, NS = info.num_cores, info.num_subcores
    NW = NC * NS

    w_idx = words.reshape(NW, B // NW)
    c_idx = contexts.T.reshape(NW, N // NW)
    V0 = (V // _TS) * _TS
    ctab = _sc_transpose(V, D, NC, NS)(c_embedding.T, c_embedding[V0:])
    w_out = _sc_gather_w(B, D, V, NC, NS)(w_idx, w_embedding)
    c_out = _sc_gather_c(N, D, NC, NS)(c_idx, ctab)
    return w_out, c_out.reshape(CTX, B, D).transpose(1, 0, 2)


# speedup vs baseline: 2.6418x; 2.6418x over previous
import functools

import jax
import jax.numpy as jnp
from jax import lax
from jax.experimental import pallas as pl
from jax.experimental.pallas import tpu as pltpu
from jax.experimental.pallas import tpu_sc as plsc

_CCH = 256
_L = 16
_TS = 128

_PARAMS = dict(use_tc_tiling_on_sc=True, needs_layout_passes=False)


def _sc_gather_c(N, D, NC, NS):
    NW = NC * NS
    bc = N // NW
    n_ch = bc // _CCH
    assert bc % _CCH == 0 and n_ch % 2 == 0 and n_ch >= 4
    mesh = plsc.VectorSubcoreMesh(core_axis_name="c", subcore_axis_name="s")

    @functools.partial(
        pl.kernel,
        out_type=jax.ShapeDtypeStruct((N, D), jnp.float32),
        mesh=mesh,
        compiler_params=pltpu.CompilerParams(**_PARAMS),
        scratch_types=[
            pltpu.VMEM((bc,), jnp.int32),
            pltpu.VMEM((2, _CCH, D), jnp.float32),
            pltpu.SemaphoreType.DMA,
            pltpu.SemaphoreType.DMA,
        ],
    )
    def body(cidx_hbm, ctab, c_out, cidx_v, rows_v, sem0, sem1):
        wid = lax.axis_index("s") * NC + lax.axis_index("c")
        pltpu.sync_copy(cidx_hbm.at[wid], cidx_v)
        c_base = wid * bc
        sems = (sem0, sem1)

        def issue(chunk, buf):
            def blk(j, carry):
                v16 = cidx_v[pl.ds(chunk * _CCH + j * _L, _L)]
                for k in range(_L):
                    pltpu.async_copy(
                        ctab.at[pl.ds(v16[k], 1)],
                        rows_v.at[buf, pl.ds(j * _L + k, 1)],
                        sems[buf],
                    )
                return carry

            lax.fori_loop(0, _CCH // _L, blk, 0)

        def drain_write(chunk, buf):
            pltpu.make_async_copy(
                c_out.at[pl.ds(0, _CCH)], rows_v.at[buf], sems[buf]
            ).wait()
            pltpu.sync_copy(
                rows_v.at[buf], c_out.at[pl.ds(c_base + chunk * _CCH, _CCH)]
            )

        issue(0, 0)
        issue(1, 1)

        def step(p, carry):
            drain_write(2 * p, 0)
            issue(2 * p + 2, 0)
            drain_write(2 * p + 1, 1)
            issue(2 * p + 3, 1)
            return carry

        lax.fori_loop(0, n_ch // 2 - 1, step, 0)
        drain_write(n_ch - 2, 0)
        drain_write(n_ch - 1, 1)

    return body


def _sc_gather_w(B, D, V, NC, NS):
    NW = NC * NS
    bw = B // NW
    assert bw % _L == 0
    mesh = plsc.VectorSubcoreMesh(core_axis_name="c", subcore_axis_name="s")

    @functools.partial(
        pl.kernel,
        out_type=jax.ShapeDtypeStruct((B, D), jnp.float32),
        mesh=mesh,
        compiler_params=pltpu.CompilerParams(**_PARAMS),
        scratch_types=[
            pltpu.VMEM((bw,), jnp.int32),
            pltpu.VMEM((bw, D), jnp.float32),
            pltpu.SemaphoreType.DMA,
        ],
    )
    def body(widx_hbm, wtab, w_out, widx_v, rows_v, sem):
        wid = lax.axis_index("s") * NC + lax.axis_index("c")
        pltpu.sync_copy(widx_hbm.at[wid], widx_v)

        def issue(j, carry):
            v16 = widx_v[pl.ds(j * _L, _L)]
            for k in range(_L):
                pltpu.async_copy(
                    wtab.at[pl.ds(v16[k], 1)],
                    rows_v.at[pl.ds(j * _L + k, 1)],
                    sem,
                )
            return carry

        lax.fori_loop(0, bw // _L, issue, 0)
        pltpu.make_async_copy(w_out.at[pl.ds(0, bw)], rows_v, sem).wait()
        pltpu.sync_copy(rows_v, w_out.at[pl.ds(wid * bw, bw)])

    return body


def kernel(words, contexts, w_embedding, c_embedding):
    (B,) = words.shape
    _, CTX = contexts.shape
    V, D = w_embedding.shape
    N = B * CTX
    info = plsc.get_sparse_core_info()
    NC, NS = info.num_cores, info.num_subcores
    NW = NC * NS

    w_idx = words.reshape(NW, B // NW)
    c_idx = contexts.T.reshape(NW, N // NW)
    w_out = _sc_gather_w(B, D, V, NC, NS)(w_idx, w_embedding)
    c_out = _sc_gather_c(N, D, NC, NS)(c_idx, c_embedding)
    return w_out, c_out.reshape(CTX, B, D).transpose(1, 0, 2)
